# baseline (device time: 515651 ns/iter reference)
import jax
import jax.numpy as jnp
from jax import lax
from jax.experimental import pallas as pl
from jax.experimental.pallas import tpu as pltpu

N_DEV = 4
S_LOC = 2048
S_HALF = S_LOC // 2
D = 1024
HQ = 8
DH = 128
QB = 256
NQB = S_LOC // QB
SCALE = 0.08838834764831843

FROM_LEFT = 0
FROM_RIGHT = 1
DIAG = 2


def _ring_attn_body(q_ref, kv_ref, out_ref, acc_ref, ml_ref, comm_ref,
                    send_sems, recv_sems):
    my = lax.axis_index("i")
    left = (my - 1) % N_DEV
    right = (my + 1) % N_DEV

    barrier_sem = pltpu.get_barrier_semaphore()
    for nbr in [left, right]:
        pl.semaphore_signal(
            barrier_sem, inc=1,
            device_id=(nbr,), device_id_type=pl.DeviceIdType.MESH,
        )
    pl.semaphore_wait(barrier_sem, 2)

    def chunk_update(k_src, v_src, first):
        def head_body(h, _):
            hds = pl.ds(h * DH, DH)
            k_h = k_src[:, hds]
            v_h = v_src[:, hds]

            def qb_body(qb, _):
                qsl = pl.ds(qb * QB, QB)
                q_blk = q_ref[qsl, hds]
                s_t = lax.dot_general(
                    k_h, q_blk, (((1,), (1,)), ((), ())),
                    preferred_element_type=jnp.float32,
                )
                if first:
                    m_new = jnp.max(s_t, axis=0, keepdims=True)
                    p = jnp.exp(s_t - m_new)
                    l_new = jnp.sum(p, axis=0, keepdims=True)
                    acc_new = lax.dot_general(
                        v_h, p.astype(jnp.bfloat16),
                        (((0,), (0,)), ((), ())),
                        preferred_element_type=jnp.float32,
                    )
                else:
                    m_old = ml_ref[h, 0:1, qsl]
                    l_old = ml_ref[h, 1:2, qsl]
                    m_new = jnp.maximum(
                        m_old, jnp.max(s_t, axis=0, keepdims=True))
                    alpha = jnp.exp(m_old - m_new)
                    p = jnp.exp(s_t - m_new)
                    l_new = l_old * alpha + jnp.sum(p, axis=0, keepdims=True)
                    acc_new = acc_ref[h, :, qsl] * alpha + lax.dot_general(
                        v_h, p.astype(jnp.bfloat16),
                        (((0,), (0,)), ((), ())),
                        preferred_element_type=jnp.float32,
                    )
                ml_ref[h, 0:1, qsl] = m_new
                ml_ref[h, 1:2, qsl] = l_new
                acc_ref[h, :, qsl] = acc_new
                return 0

            lax.fori_loop(0, NQB, qb_body, 0)
            return 0

        lax.fori_loop(0, HQ, head_body, 0)

    def remote_copy(src, dst, sem_idx, target):
        return pltpu.make_async_remote_copy(
            src_ref=src, dst_ref=dst,
            send_sem=send_sems.at[sem_idx],
            recv_sem=recv_sems.at[sem_idx],
            device_id=(target,),
            device_id_type=pl.DeviceIdType.MESH,
        )

    a_right = remote_copy(kv_ref, comm_ref.at[FROM_LEFT], 0, right)
    a_left = remote_copy(kv_ref, comm_ref.at[FROM_RIGHT], 1, left)
    a_right.start()
    a_left.start()

    chunk_update(kv_ref.at[0], kv_ref.at[1], first=True)

    a_right.wait_recv()
    b_right = remote_copy(
        comm_ref.at[FROM_LEFT, :, pl.ds(0, S_HALF)],
        comm_ref.at[DIAG, :, pl.ds(0, S_HALF)], 2, right)
    b_right.start()
    chunk_update(comm_ref.at[FROM_LEFT, 0], comm_ref.at[FROM_LEFT, 1],
                 first=False)

    a_left.wait_recv()
    b_left = remote_copy(
        comm_ref.at[FROM_RIGHT, :, pl.ds(S_HALF, S_HALF)],
        comm_ref.at[DIAG, :, pl.ds(S_HALF, S_HALF)], 3, left)
    b_left.start()
    chunk_update(comm_ref.at[FROM_RIGHT, 0], comm_ref.at[FROM_RIGHT, 1],
                 first=False)

    b_right.wait_recv()
    b_left.wait_recv()
    chunk_update(comm_ref.at[DIAG, 0], comm_ref.at[DIAG, 1], first=False)

    a_right.wait_send()
    a_left.wait_send()
    b_right.wait_send()
    b_left.wait_send()

    eye = (lax.broadcasted_iota(jnp.int32, (DH, DH), 0)
           == lax.broadcasted_iota(jnp.int32, (DH, DH), 1)).astype(jnp.float32)
    for h in range(HQ):
        ctx_t = acc_ref[h] / ml_ref[h, 1:2, :]
        out_blk = lax.dot_general(
            ctx_t, eye, (((0,), (0,)), ((), ())),
            preferred_element_type=jnp.float32,
        )
        out_ref[:, h * DH:(h + 1) * DH] = out_blk.astype(jnp.bfloat16)


def _ring_attn(q, kv):
    out, _, _ = pl.pallas_call(
        _ring_attn_body,
        out_shape=[
            jax.ShapeDtypeStruct((S_LOC, D), jnp.bfloat16),
            jax.ShapeDtypeStruct((HQ, DH, S_LOC), jnp.float32),
            jax.ShapeDtypeStruct((HQ, 2, S_LOC), jnp.float32),
        ],
        in_specs=[pl.BlockSpec(memory_space=pltpu.VMEM)] * 2,
        out_specs=[pl.BlockSpec(memory_space=pltpu.VMEM)] * 3,
        scratch_shapes=[
            pltpu.VMEM((3, 2, S_LOC, D), jnp.bfloat16),
            pltpu.SemaphoreType.DMA((4,)),
            pltpu.SemaphoreType.DMA((4,)),
        ],
        compiler_params=pltpu.CompilerParams(
            collective_id=0,
            vmem_limit_bytes=56 * 1024 * 1024,
        ),
    )(q, kv)
    return out


def kernel(x, Wq, Wk, Wv, Wo):
    xb = x[0].astype(jnp.bfloat16)
    q = jnp.dot(xb, Wq.astype(jnp.bfloat16), preferred_element_type=jnp.float32)
    k = jnp.dot(xb, Wk.astype(jnp.bfloat16), preferred_element_type=jnp.float32)
    v = jnp.dot(xb, Wv.astype(jnp.bfloat16), preferred_element_type=jnp.bfloat16)

    my = lax.axis_index("i")
    pos = (my * S_LOC + jnp.arange(S_LOC)).astype(jnp.float32)
    inv = 1.0 / (10000.0 ** (jnp.arange(0, DH, 2, dtype=jnp.float32) / DH))
    ang = pos[:, None] * inv[None, :]
    cos = jnp.repeat(jnp.cos(ang), 2, axis=-1)
    sin = jnp.repeat(jnp.sin(ang), 2, axis=-1)

    def rope(t):
        t4 = t.reshape(S_LOC, HQ, DH // 2, 2)
        t_r = jnp.stack([-t4[..., 1], t4[..., 0]], axis=-1).reshape(S_LOC, HQ, DH)
        th = t.reshape(S_LOC, HQ, DH)
        return (th * cos[:, None, :] + t_r * sin[:, None, :]).reshape(S_LOC, D)

    qb = (rope(q) * SCALE).astype(jnp.bfloat16)
    kv = jnp.stack([rope(k).astype(jnp.bfloat16), v])

    ctx = _ring_attn(qb, kv)

    out = jnp.dot(ctx, Wo.astype(jnp.bfloat16),
                  preferred_element_type=jnp.float32)
    return out[None]
